# baseline (speedup 1.0000x reference)
import functools

import jax
import jax.numpy as jnp
from jax import lax
from jax.experimental import pallas as pl
from jax.experimental.pallas import tpu as pltpu
from jax.experimental.pallas import tpu_sc as plsc

B = 2
N = 10000
E = 320000
D = 128
R = B * (N + E)
NC, NS = 2, 16
NW = NC * NS
W = 120
WPT = 174
RPAD = NW * WPT * W
HPT = WPT // 6


def _gunpool_sc(xh, idx0, idx1):
    mesh = plsc.VectorSubcoreMesh(core_axis_name="c", subcore_axis_name="s")

    @functools.partial(
        pl.kernel,
        out_type=jax.ShapeDtypeStruct((R, D), jnp.float32),
        mesh=mesh,
        scratch_types=[
            pltpu.VMEM((WPT * W,), jnp.int32),
            pltpu.VMEM((WPT * W,), jnp.int32),
            pltpu.VMEM((W, D), jnp.float32),
            pltpu.VMEM((W, D), jnp.float32),
            pltpu.VMEM((W, D), jnp.float32),
            pltpu.VMEM((W, D), jnp.float32),
            pltpu.VMEM((W, D), jnp.float32),
            pltpu.SemaphoreType.DMA,
            pltpu.SemaphoreType.DMA,
            pltpu.SemaphoreType.DMA,
            pltpu.SemaphoreType.DMA,
            pltpu.SemaphoreType.DMA,
            pltpu.SemaphoreType.DMA,
            pltpu.SemaphoreType.DMA,
            pltpu.SemaphoreType.DMA,
        ],
    )
    def k(x_hbm, i0_hbm, i1_hbm, out_hbm, i0_all, i1_all,
          b0_0, b0_1, b0_2, b1_0, b1_1,
          g0_0, g0_1, g0_2, g1_0, g1_1, o_0, o_1, o_2):
        b0s = [b0_0, b0_1, b0_2]
        b1s = [b1_0, b1_1]
        g0s = [g0_0, g0_1, g0_2]
        g1s = [g1_0, g1_1]
        oss = [o_0, o_1, o_2]

        wid = lax.axis_index("s") * NC + lax.axis_index("c")
        tile_base = wid * (WPT * W)

        pltpu.sync_copy(i0_hbm.at[pl.ds(tile_base, WPT * W)], i0_all)
        pltpu.sync_copy(i1_hbm.at[pl.ds(tile_base, WPT * W)], i1_all)

        def gather_pair(t, p):
            pltpu.async_copy(
                x_hbm.at[i0_all.at[pl.ds(t * W, W)]], b0s[p % 3], g0s[p % 3])
            pltpu.async_copy(
                x_hbm.at[i1_all.at[pl.ds(t * W, W)]], b1s[p % 2], g1s[p % 2])

        def wait_pair(t, p):
            pltpu.make_async_copy(
                x_hbm.at[i0_all.at[pl.ds(t * W, W)]], b0s[p % 3],
                g0s[p % 3]).wait()
            pltpu.make_async_copy(
                x_hbm.at[i1_all.at[pl.ds(t * W, W)]], b1s[p % 2],
                g1s[p % 2]).wait()

        def wait_store(t, p):
            base = tile_base + t * W

            @pl.when((t >= 0) & (base < R))
            def _():
                pltpu.make_async_copy(
                    b0s[p % 3], out_hbm.at[pl.ds(base, W)], oss[p % 3]).wait()

        gather_pair(0, 0)

        @pl.loop(0, HPT)
        def _(kk):
            t0 = 6 * kk
            for p in range(6):
                t = t0 + p
                wait_store(t - 2, p + 1)

                @pl.when(t + 1 < WPT)
                def _():
                    gather_pair(t + 1, p + 1)

                wait_pair(t, p)

                @pl.loop(0, W)
                def _(r):
                    for c in range(0, D, 16):
                        plsc.addupdate(
                            b0s[p % 3].at[r, pl.ds(c, 16)],
                            b1s[p % 2][r, pl.ds(c, 16)])

                base = tile_base + t * W

                @pl.when(base < R)
                def _():
                    pltpu.async_copy(
                        b0s[p % 3], out_hbm.at[pl.ds(base, W)], oss[p % 3])

        wait_store(WPT - 2, WPT - 2)
        wait_store(WPT - 1, WPT - 1)

    return k(xh, idx0, idx1)


def kernel(inputs, unpool_idx):
    u0 = unpool_idx[:, 0].astype(jnp.int32)
    u1 = unpool_idx[:, 1].astype(jnp.int32)
    ar = jnp.arange(N, dtype=jnp.int32)
    pad = jnp.zeros((RPAD - R,), jnp.int32)
    idx0 = jnp.concatenate([ar, u0, ar + N, u0 + N, pad])
    idx1 = jnp.concatenate([ar, u1, ar + N, u1 + N, pad])
    xh = (inputs * 0.5).reshape(B * N, D)
    out = _gunpool_sc(xh, idx0, idx1)
    return out.reshape(B, N + E, D)


# speedup vs baseline: 2.3765x; 2.3765x over previous
import functools

import jax
import jax.numpy as jnp
from jax import lax
from jax.experimental import pallas as pl
from jax.experimental.pallas import tpu as pltpu
from jax.experimental.pallas import tpu_sc as plsc

B = 2
N = 10000
E = 320000
D = 128
R = B * (N + E)
NC, NS = 2, 16
NW = NC * NS
W = 120
WPT = (R + NW * W - 1) // (NW * W)
RPAD = NW * WPT * W
HPT = WPT // 2


def _gunpool_sc(xh, idx0, idx1):
    mesh = plsc.VectorSubcoreMesh(core_axis_name="c", subcore_axis_name="s")

    @functools.partial(
        pl.kernel,
        out_type=jax.ShapeDtypeStruct((R, D), jnp.float32),
        mesh=mesh,
        scratch_types=[
            pltpu.VMEM((WPT * W,), jnp.int32),
            pltpu.VMEM((WPT * W,), jnp.int32),
            pltpu.VMEM((W, D), jnp.float32),
            pltpu.VMEM((W, D), jnp.float32),
            pltpu.VMEM((W, D), jnp.float32),
            pltpu.VMEM((W, D), jnp.float32),
            pltpu.SemaphoreType.DMA,
            pltpu.SemaphoreType.DMA,
            pltpu.SemaphoreType.DMA,
            pltpu.SemaphoreType.DMA,
        ],
    )
    def k(x_hbm, i0_hbm, i1_hbm, out_hbm, i0_all, i1_all, b0a, b1a, b0b, b1b,
          sa0, sa1, sb0, sb1):
        wid = lax.axis_index("s") * NC + lax.axis_index("c")
        slab = wid + (wid == 16).astype(jnp.int32) - (wid == 17).astype(jnp.int32)
        tile_base = slab * (WPT * W)

        pltpu.sync_copy(i0_hbm.at[pl.ds(tile_base, WPT * W)], i0_all)
        pltpu.sync_copy(i1_hbm.at[pl.ds(tile_base, WPT * W)], i1_all)

        def live(s):
            return tile_base + s * W < R

        def gather(s, d0, d1, s0, s1):
            @pl.when(live(s))
            def _():
                pltpu.async_copy(x_hbm.at[i0_all.at[pl.ds(s * W, W)]], d0, s0)
                pltpu.async_copy(x_hbm.at[i1_all.at[pl.ds(s * W, W)]], d1, s1)

        def wait(s, d0, d1, s0, s1):
            @pl.when(live(s))
            def _():
                pltpu.make_async_copy(
                    x_hbm.at[i0_all.at[pl.ds(s * W, W)]], d0, s0).wait()
                pltpu.make_async_copy(
                    x_hbm.at[i1_all.at[pl.ds(s * W, W)]], d1, s1).wait()

        def accum_store(s, d0, d1):
            @pl.when(live(s))
            def _():
                @pl.loop(0, W)
                def _(r):
                    for c in range(0, D, 16):
                        plsc.addupdate(d0.at[r, pl.ds(c, 16)], d1[r, pl.ds(c, 16)])

                pltpu.sync_copy(d0, out_hbm.at[pl.ds(tile_base + s * W, W)])

        gather(0, b0a, b1a, sa0, sa1)

        @pl.loop(0, HPT)
        def _(kk):
            s = 2 * kk
            gather(s + 1, b0b, b1b, sb0, sb1)
            wait(s, b0a, b1a, sa0, sa1)
            accum_store(s, b0a, b1a)

            @pl.when(kk < HPT - 1)
            def _():
                gather(s + 2, b0a, b1a, sa0, sa1)

            wait(s + 1, b0b, b1b, sb0, sb1)
            accum_store(s + 1, b0b, b1b)

    return k(xh, idx0, idx1)


def kernel(inputs, unpool_idx):
    u0 = unpool_idx[:, 0].astype(jnp.int32)
    u1 = unpool_idx[:, 1].astype(jnp.int32)
    ar = jnp.arange(N, dtype=jnp.int32)
    pad = jnp.zeros((RPAD - R,), jnp.int32)
    idx0 = jnp.concatenate([ar, u0, ar + N, u0 + N, pad])
    idx1 = jnp.concatenate([ar, u1, ar + N, u1 + N, pad])
    xh = (inputs * 0.5).reshape(B * N, D)
    out = _gunpool_sc(xh, idx0, idx1)
    return out.reshape(B, N + E, D)


# speedup vs baseline: 2.4154x; 1.0164x over previous
import functools

import jax
import jax.numpy as jnp
from jax import lax
from jax.experimental import pallas as pl
from jax.experimental.pallas import tpu as pltpu
from jax.experimental.pallas import tpu_sc as plsc

B = 2
N = 10000
E = 320000
D = 128
R = B * (N + E)
NC, NS = 2, 16
NW = NC * NS
W = 120
WPT = (R + NW * W - 1) // (NW * W)
RPAD = NW * WPT * W
HPT = WPT // 2


def _gunpool_sc(xh, idx0, idx1):
    mesh = plsc.VectorSubcoreMesh(core_axis_name="c", subcore_axis_name="s")

    @functools.partial(
        pl.kernel,
        out_type=jax.ShapeDtypeStruct((R, D), jnp.float32),
        mesh=mesh,
        scratch_types=[
            pltpu.VMEM((WPT * W,), jnp.int32),
            pltpu.VMEM((WPT * W,), jnp.int32),
            pltpu.VMEM((W, D), jnp.float32),
            pltpu.VMEM((W, D), jnp.float32),
            pltpu.VMEM((W, D), jnp.float32),
            pltpu.VMEM((W, D), jnp.float32),
            pltpu.VMEM((W, D), jnp.float32),
            pltpu.SemaphoreType.DMA,
            pltpu.SemaphoreType.DMA,
            pltpu.SemaphoreType.DMA,
            pltpu.SemaphoreType.DMA,
            pltpu.SemaphoreType.DMA,
        ],
    )
    def k(x_hbm, i0_hbm, i1_hbm, out_hbm, i0_all, i1_all, b0a, b1a, b0b, b1b,
          stb, sa0, sa1, sb0, sb1, sst):
        wid = lax.axis_index("s") * NC + lax.axis_index("c")
        slab = wid + (wid == 16).astype(jnp.int32) - (wid == 17).astype(jnp.int32)
        tile_base = slab * (WPT * W)

        pltpu.sync_copy(i0_hbm.at[pl.ds(tile_base, WPT * W)], i0_all)
        pltpu.sync_copy(i1_hbm.at[pl.ds(tile_base, WPT * W)], i1_all)

        def live(s):
            return tile_base + s * W < R

        def gather(s, d0, d1, s0, s1):
            @pl.when(live(s))
            def _():
                pltpu.async_copy(x_hbm.at[i0_all.at[pl.ds(s * W, W)]], d0, s0)
                pltpu.async_copy(x_hbm.at[i1_all.at[pl.ds(s * W, W)]], d1, s1)

        def wait(s, d0, d1, s0, s1):
            @pl.when(live(s))
            def _():
                pltpu.make_async_copy(
                    x_hbm.at[i0_all.at[pl.ds(s * W, W)]], d0, s0).wait()
                pltpu.make_async_copy(
                    x_hbm.at[i1_all.at[pl.ds(s * W, W)]], d1, s1).wait()

        def wait_store(s):
            pltpu.make_async_copy(
                stb, out_hbm.at[pl.ds(tile_base + s * W, W)], sst).wait()

        def accum_store(s, d0, d1, first):
            @pl.when(live(s))
            def _():
                if not first:
                    wait_store(s - 1)

                @pl.loop(0, W)
                def _(r):
                    for c in range(0, D, 16):
                        stb[r, pl.ds(c, 16)] = (
                            d0[r, pl.ds(c, 16)] + d1[r, pl.ds(c, 16)])

                pltpu.async_copy(
                    stb, out_hbm.at[pl.ds(tile_base + s * W, W)], sst)

        gather(0, b0a, b1a, sa0, sa1)

        @pl.loop(0, HPT)
        def _(kk):
            s = 2 * kk
            gather(s + 1, b0b, b1b, sb0, sb1)
            wait(s, b0a, b1a, sa0, sa1)

            @pl.when(kk == 0)
            def _():
                accum_store(s, b0a, b1a, first=True)

            @pl.when(kk > 0)
            def _():
                accum_store(s, b0a, b1a, first=False)

            @pl.when(kk < HPT - 1)
            def _():
                gather(s + 2, b0a, b1a, sa0, sa1)

            wait(s + 1, b0b, b1b, sb0, sb1)
            accum_store(s + 1, b0b, b1b, first=False)

        live_wins = jnp.clip((R - tile_base + W - 1) // W, 0, WPT)
        wait_store(live_wins - 1)

    return k(xh, idx0, idx1)


def kernel(inputs, unpool_idx):
    u0 = unpool_idx[:, 0].astype(jnp.int32)
    u1 = unpool_idx[:, 1].astype(jnp.int32)
    ar = jnp.arange(N, dtype=jnp.int32)
    pad = jnp.zeros((RPAD - R,), jnp.int32)
    idx0 = jnp.concatenate([ar, u0, ar + N, u0 + N, pad])
    idx1 = jnp.concatenate([ar, u1, ar + N, u1 + N, pad])
    xh = (inputs * 0.5).reshape(B * N, D)
    out = _gunpool_sc(xh, idx0, idx1)
    return out.reshape(B, N + E, D)
